# baseline (speedup 1.0000x reference)
import functools

import jax
import jax.numpy as jnp
from jax import lax
from jax.experimental import pallas as pl
from jax.experimental.pallas import tpu as pltpu
from jax.experimental.pallas import tpu_sc as plsc

N = 10000
C = 128
E = 320000
NE = 2 * E
GW = 10240
GWW = GW // 2

NC, NS = 2, 16
NW = NC * NS
EPW = E // NW
CH = 2000
NCHUNK = EPW // CH
VL = 16
K_RANK = 576000


def _softmax_norms_body(x_ref, p_ref, n_ref):
    x = x_ref[0]
    m = jnp.max(x, axis=-1, keepdims=True)
    e = jnp.exp(x - m)
    s = jnp.sum(e, axis=-1, keepdims=True)
    p = e / s
    p_ref[0] = p
    n_ref[0, 0, 0] = jnp.sum(p * p, axis=-1)


def _softmax_norms(logits):
    return pl.pallas_call(
        _softmax_norms_body,
        grid=(2,),
        in_specs=[pl.BlockSpec((1, N, C), lambda g: (g, 0, 0))],
        out_specs=[
            pl.BlockSpec((1, N, C), lambda g: (g, 0, 0)),
            pl.BlockSpec((1, 1, 1, N), lambda g: (g, 0, 0, 0)),
        ],
        out_shape=[
            jax.ShapeDtypeStruct((2, N, C), jnp.float32),
            jax.ShapeDtypeStruct((2, 1, 1, N), jnp.float32),
        ],
    )(logits)


BI, BJ = 1000, 2048


def _gram_body(a_ref, b_ref, o_ref):
    i = pl.program_id(0)
    j = pl.program_id(1)

    @pl.when(j * BJ + BJ > i * BI)
    def _():
        res = lax.dot_general(
            a_ref[...].astype(jnp.bfloat16), b_ref[...].astype(jnp.bfloat16),
            (((1,), (1,)), ((), ())),
            preferred_element_type=jnp.float32)
        h = BJ // 2
        ue = lax.bitcast_convert_type(res[:, :h], jnp.int32)
        uo = lax.bitcast_convert_type(res[:, h:], jnp.int32)
        lo = lax.shift_right_logical(ue + jnp.int32(0x8000), 16)
        hi = lax.bitwise_and(uo + jnp.int32(0x8000), jnp.int32(-65536))
        words = lax.bitwise_or(lo, hi)
        for s in range(BJ // 2 // C):
            o_ref[:, s, :] = words[:, s * C:(s + 1) * C]


def _gram(p_pad):
    def omap(i, j):
        jmin = (i * BI) // BJ
        return (i, jnp.maximum(j, jmin), 0)

    nb = BJ // 2 // C

    return pl.pallas_call(
        _gram_body,
        grid=(N // BI, GW // BJ),
        in_specs=[
            pl.BlockSpec((BI, C), lambda i, j: (i, 0)),
            pl.BlockSpec((BJ, C), lambda i, j: (j, 0)),
        ],
        out_specs=pl.BlockSpec((BI, nb, C), omap),
        out_shape=jax.ShapeDtypeStruct((N, GWW // C, C), jnp.int32),
    )(p_pad, p_pad)


def _sc_edges_body(g_hbm, ei_hbm, ej_hbm, nrm_hbm, dep_hbm, lbl_hbm,
                   pd2_hbm, dd_hbm, cs_hbm,
                   iv, jv, fidx, pshift, gv, nv, dv, lv,
                   o_pd2, o_dd, o_cs, sem):
    wid = lax.axis_index("s") * NC + lax.axis_index("c")

    pltpu.sync_copy(nrm_hbm, nv)
    pltpu.sync_copy(dep_hbm, dv)
    pltpu.sync_copy(lbl_hbm, lv)

    def chunk_body(cix, carry):
        base = wid * EPW + cix * CH
        pltpu.sync_copy(ei_hbm.at[pl.ds(base, CH)], iv)
        pltpu.sync_copy(ej_hbm.at[pl.ds(base, CH)], jv)

        def pass1(k, carry2):
            sl = pl.ds(k * VL, VL)
            ivec = iv[sl]
            jvec = jv[sl]
            mn = jnp.minimum(ivec, jvec)
            mx = jnp.maximum(ivec, jvec)
            w = (lax.shift_left(lax.shift_right_logical(mx, 11), 10)
                 + lax.bitwise_and(mx, jnp.int32(1023)))
            fidx[sl] = mn * GWW + w
            pshift[sl] = lax.shift_left(
                lax.bitwise_and(lax.shift_right_logical(mx, 10),
                                jnp.int32(1)), 4)
            di = plsc.load_gather(dv, [ivec])
            dj = plsc.load_gather(dv, [jvec])
            o_dd[sl] = jnp.abs(di - dj)
            li = plsc.load_gather(lv, [ivec])
            lj = plsc.load_gather(lv, [jvec])
            o_cs[sl] = jnp.where(li == lj, jnp.float32(1.0), jnp.float32(0.0))
            ni = plsc.load_gather(nv, [ivec])
            nj = plsc.load_gather(nv, [jvec])
            o_pd2[sl] = ni + nj
            return carry2

        lax.fori_loop(0, CH // VL, pass1, 0)

        cps = []
        for s in range(15):
            cps.append(pltpu.async_copy(
                g_hbm.at[fidx.at[pl.ds(s * 128, 128)]],
                gv.at[pl.ds(s * 128, 128)], sem))
        cps.append(pltpu.async_copy(
            g_hbm.at[fidx.at[pl.ds(1920, 80)]],
            gv.at[pl.ds(1920, 80)], sem))
        for cp in cps:
            cp.wait()

        def pass2(k, carry2):
            sl = pl.ds(k * VL, VL)
            bits = lax.shift_left(
                lax.shift_right_logical(gv[sl], pshift[sl]), 16)
            gval = plsc.bitcast(bits, jnp.float32)
            pd2 = o_pd2[sl] - jnp.float32(2.0) * gval
            o_pd2[sl] = jnp.maximum(pd2, jnp.float32(0.0))
            return carry2

        lax.fori_loop(0, CH // VL, pass2, 0)

        pltpu.sync_copy(o_pd2, pd2_hbm.at[pl.ds(base, CH)])
        pltpu.sync_copy(o_dd, dd_hbm.at[pl.ds(base, CH)])
        pltpu.sync_copy(o_cs, cs_hbm.at[pl.ds(base, CH)])
        return carry

    lax.fori_loop(0, NCHUNK, chunk_body, 0)


def _sc_edges(g_flat, ei, ej, nrm, dep, lbl):
    mesh = plsc.VectorSubcoreMesh(core_axis_name="c", subcore_axis_name="s")
    f32 = jnp.float32
    kern = functools.partial(
        pl.kernel,
        mesh=mesh,
        compiler_params=pltpu.CompilerParams(needs_layout_passes=False),
        out_type=[
            jax.ShapeDtypeStruct((E,), f32),
            jax.ShapeDtypeStruct((E,), f32),
            jax.ShapeDtypeStruct((E,), f32),
        ],
        scratch_types=[
            pltpu.VMEM((CH,), jnp.int32),
            pltpu.VMEM((CH,), jnp.int32),
            pltpu.VMEM((CH,), jnp.int32),
            pltpu.VMEM((CH,), jnp.int32),
            pltpu.VMEM((CH,), jnp.int32),
            pltpu.VMEM((N,), f32),
            pltpu.VMEM((N,), f32),
            pltpu.VMEM((N,), jnp.int32),
            pltpu.VMEM((CH,), f32),
            pltpu.VMEM((CH,), f32),
            pltpu.VMEM((CH,), f32),
            pltpu.SemaphoreType.DMA,
        ],
    )(_sc_edges_body)
    return kern(g_flat, ei, ej, nrm, dep, lbl)


def _finalize_body(pd2a_ref, pd2b_ref, dda_ref, ddb_ref, csa_ref, csb_ref,
                   raw_ref, o_ref):
    pda = jnp.sqrt(pd2a_ref[...])
    pdb = jnp.sqrt(pd2b_ref[...])

    def masked_mean(sa, sb, ca, cb):
        cnt = jnp.sum(ca) + jnp.sum(cb)
        s = jnp.sum(sa) + jnp.sum(sb)
        return jnp.where(cnt > 0, s / jnp.maximum(cnt, 1.0), 0.0)

    csa = csa_ref[...]
    csb = csb_ref[...]
    p_cluster = masked_mean(pda * csa, pdb * csb, csa, csb)
    ma = (dda_ref[...] < 3.0).astype(jnp.float32)
    mb = (ddb_ref[...] < 3.0).astype(jnp.float32)
    p_depth = masked_mean(pda * ma, pdb * mb, ma, mb)

    bits_a = lax.bitcast_convert_type(pda, jnp.int32)
    bits_b = lax.bitcast_convert_type(pdb, jnp.int32)

    def bs_body(_, lohi):
        lo, hi = lohi
        mid = (lo + hi) // 2
        cnt = (jnp.sum((bits_a <= mid).astype(jnp.int32))
               + jnp.sum((bits_b <= mid).astype(jnp.int32)))
        return jnp.where(cnt >= K_RANK, lo, mid + 1), jnp.where(
            cnt >= K_RANK, mid, hi)

    lo, hi = lax.fori_loop(0, 31, bs_body,
                           (jnp.int32(0), jnp.int32(0x7F800000)))
    v1 = lax.bitcast_convert_type(hi, jnp.float32)
    ja = (pda > v1).astype(jnp.float32)
    jb = (pdb > v1).astype(jnp.float32)
    p_jump = masked_mean(pda * ja, pdb * jb, ja, jb)

    e = jnp.exp(raw_ref[...])
    w = e / jnp.sum(e)
    total = w[0, 0] * p_cluster + w[0, 1] * p_depth + w[0, 2] * p_jump
    o_ref[...] = jnp.reshape(total, (1, 1))


def _finalize(pd2a, pd2b, dda, ddb, csa, csb, raw_pad):
    return pl.pallas_call(
        _finalize_body,
        out_shape=jax.ShapeDtypeStruct((1, 1), jnp.float32),
    )(pd2a, pd2b, dda, ddb, csa, csb, raw_pad)


def kernel(logits_src, logits_tgt, edge_index_src, edge_index_tgt,
           cluster_labels_src, cluster_labels_tgt, depth_src, depth_tgt, raw):
    logits = jnp.stack([logits_src, logits_tgt]).astype(jnp.float32)
    probs, norms = _softmax_norms(logits)
    p_pad = jnp.pad(probs, ((0, 0), (0, GW - N), (0, 0)))

    gram_a = _gram(p_pad[0])
    pd2a, dda, csa = _sc_edges(
        gram_a.reshape(N * GWW),
        edge_index_src[0].astype(jnp.int32),
        edge_index_src[1].astype(jnp.int32),
        norms.reshape(2, N)[0], depth_src.astype(jnp.float32),
        cluster_labels_src.astype(jnp.int32))
    gram_b = _gram(p_pad[1])
    pd2b, ddb, csb = _sc_edges(
        gram_b.reshape(N * GWW),
        edge_index_tgt[0].astype(jnp.int32),
        edge_index_tgt[1].astype(jnp.int32),
        norms.reshape(2, N)[1], depth_tgt.astype(jnp.float32),
        cluster_labels_tgt.astype(jnp.int32))

    raw_pad = jnp.full((1, C), -jnp.inf, jnp.float32).at[0, :3].set(
        raw.astype(jnp.float32))
    out = _finalize(pd2a.reshape(2500, 128), pd2b.reshape(2500, 128),
                    dda.reshape(2500, 128), ddb.reshape(2500, 128),
                    csa.reshape(2500, 128), csb.reshape(2500, 128), raw_pad)
    return out[0, 0]


# speedup vs baseline: 1.0085x; 1.0085x over previous
import functools

import jax
import jax.numpy as jnp
from jax import lax
from jax.experimental import pallas as pl
from jax.experimental.pallas import tpu as pltpu
from jax.experimental.pallas import tpu_sc as plsc

N = 10000
C = 128
E = 320000
NE = 2 * E
GW = 10240
GWW = GW // 2

NC, NS = 2, 16
NW = NC * NS
EPW = E // NW
CH = 2000
NCHUNK = EPW // CH
VL = 16
K_RANK = 576000


def _softmax_norms_body(x_ref, p_ref, n_ref):
    x = x_ref[0]
    m = jnp.max(x, axis=-1, keepdims=True)
    e = jnp.exp(x - m)
    s = jnp.sum(e, axis=-1, keepdims=True)
    p = e / s
    p_ref[0] = p
    n_ref[0, 0, 0] = jnp.sum(p * p, axis=-1)


def _softmax_norms(logits):
    return pl.pallas_call(
        _softmax_norms_body,
        grid=(2,),
        in_specs=[pl.BlockSpec((1, N, C), lambda g: (g, 0, 0))],
        out_specs=[
            pl.BlockSpec((1, N, C), lambda g: (g, 0, 0)),
            pl.BlockSpec((1, 1, 1, N), lambda g: (g, 0, 0, 0)),
        ],
        out_shape=[
            jax.ShapeDtypeStruct((2, N, C), jnp.float32),
            jax.ShapeDtypeStruct((2, 1, 1, N), jnp.float32),
        ],
    )(logits)


BI, BJ = 1000, 2048


def _gram_body(a_ref, b_ref, o_ref):
    i = pl.program_id(0)
    j = pl.program_id(1)

    @pl.when(j * BJ + BJ > i * BI)
    def _():
        res = lax.dot_general(
            a_ref[...], b_ref[...], (((1,), (1,)), ((), ())),
            preferred_element_type=jnp.float32)
        h = BJ // 2
        ue = lax.bitcast_convert_type(res[:, :h], jnp.int32)
        uo = lax.bitcast_convert_type(res[:, h:], jnp.int32)
        lo = lax.shift_right_logical(ue + jnp.int32(0x8000), 16)
        hi = lax.bitwise_and(uo + jnp.int32(0x8000), jnp.int32(-65536))
        words = lax.bitwise_or(lo, hi)
        for s in range(BJ // 2 // C):
            o_ref[:, s, :] = words[:, s * C:(s + 1) * C]


def _gram(p_pad):
    def omap(i, j):
        jmin = (i * BI) // BJ
        return (i, jnp.maximum(j, jmin), 0)

    nb = BJ // 2 // C

    return pl.pallas_call(
        _gram_body,
        grid=(N // BI, GW // BJ),
        in_specs=[
            pl.BlockSpec((BI, C), lambda i, j: (i, 0)),
            pl.BlockSpec((BJ, C), lambda i, j: (j, 0)),
        ],
        out_specs=pl.BlockSpec((BI, nb, C), omap),
        out_shape=jax.ShapeDtypeStruct((N, GWW // C, C), jnp.int32),
    )(p_pad, p_pad)


def _sc_edges_body(g_hbm, ei_hbm, ej_hbm, nrm_hbm, dep_hbm, lbl_hbm,
                   pd2_hbm, dd_hbm, cs_hbm,
                   iv, jv, fidx, pshift, gv, nv, dv, lv,
                   o_pd2, o_dd, o_cs, sem):
    wid = lax.axis_index("s") * NC + lax.axis_index("c")

    pltpu.sync_copy(nrm_hbm, nv)
    pltpu.sync_copy(dep_hbm, dv)
    pltpu.sync_copy(lbl_hbm, lv)

    def chunk_body(cix, carry):
        base = wid * EPW + cix * CH
        pltpu.sync_copy(ei_hbm.at[pl.ds(base, CH)], iv)
        pltpu.sync_copy(ej_hbm.at[pl.ds(base, CH)], jv)

        def pass1(k, carry2):
            sl = pl.ds(k * VL, VL)
            ivec = iv[sl]
            jvec = jv[sl]
            mn = jnp.minimum(ivec, jvec)
            mx = jnp.maximum(ivec, jvec)
            w = (lax.shift_left(lax.shift_right_logical(mx, 11), 10)
                 + lax.bitwise_and(mx, jnp.int32(1023)))
            fidx[sl] = mn * GWW + w
            pshift[sl] = lax.shift_left(
                lax.bitwise_and(lax.shift_right_logical(mx, 10),
                                jnp.int32(1)), 4)
            di = plsc.load_gather(dv, [ivec])
            dj = plsc.load_gather(dv, [jvec])
            o_dd[sl] = jnp.abs(di - dj)
            li = plsc.load_gather(lv, [ivec])
            lj = plsc.load_gather(lv, [jvec])
            o_cs[sl] = jnp.where(li == lj, jnp.float32(1.0), jnp.float32(0.0))
            ni = plsc.load_gather(nv, [ivec])
            nj = plsc.load_gather(nv, [jvec])
            o_pd2[sl] = ni + nj
            return carry2

        lax.fori_loop(0, CH // VL, pass1, 0)

        cps = []
        for s in range(15):
            cps.append(pltpu.async_copy(
                g_hbm.at[fidx.at[pl.ds(s * 128, 128)]],
                gv.at[pl.ds(s * 128, 128)], sem))
        cps.append(pltpu.async_copy(
            g_hbm.at[fidx.at[pl.ds(1920, 80)]],
            gv.at[pl.ds(1920, 80)], sem))
        for cp in cps:
            cp.wait()

        def pass2(k, carry2):
            sl = pl.ds(k * VL, VL)
            bits = lax.shift_left(
                lax.shift_right_logical(gv[sl], pshift[sl]), 16)
            gval = plsc.bitcast(bits, jnp.float32)
            pd2 = o_pd2[sl] - jnp.float32(2.0) * gval
            o_pd2[sl] = jnp.maximum(pd2, jnp.float32(0.0))
            return carry2

        lax.fori_loop(0, CH // VL, pass2, 0)

        pltpu.sync_copy(o_pd2, pd2_hbm.at[pl.ds(base, CH)])
        pltpu.sync_copy(o_dd, dd_hbm.at[pl.ds(base, CH)])
        pltpu.sync_copy(o_cs, cs_hbm.at[pl.ds(base, CH)])
        return carry

    lax.fori_loop(0, NCHUNK, chunk_body, 0)


def _sc_edges(g_flat, ei, ej, nrm, dep, lbl):
    mesh = plsc.VectorSubcoreMesh(core_axis_name="c", subcore_axis_name="s")
    f32 = jnp.float32
    kern = functools.partial(
        pl.kernel,
        mesh=mesh,
        compiler_params=pltpu.CompilerParams(needs_layout_passes=False),
        out_type=[
            jax.ShapeDtypeStruct((E,), f32),
            jax.ShapeDtypeStruct((E,), f32),
            jax.ShapeDtypeStruct((E,), f32),
        ],
        scratch_types=[
            pltpu.VMEM((CH,), jnp.int32),
            pltpu.VMEM((CH,), jnp.int32),
            pltpu.VMEM((CH,), jnp.int32),
            pltpu.VMEM((CH,), jnp.int32),
            pltpu.VMEM((CH,), jnp.int32),
            pltpu.VMEM((N,), f32),
            pltpu.VMEM((N,), f32),
            pltpu.VMEM((N,), jnp.int32),
            pltpu.VMEM((CH,), f32),
            pltpu.VMEM((CH,), f32),
            pltpu.VMEM((CH,), f32),
            pltpu.SemaphoreType.DMA,
        ],
    )(_sc_edges_body)
    return kern(g_flat, ei, ej, nrm, dep, lbl)


def _finalize_body(pd2a_ref, pd2b_ref, dda_ref, ddb_ref, csa_ref, csb_ref,
                   raw_ref, o_ref):
    pda = jnp.sqrt(pd2a_ref[...])
    pdb = jnp.sqrt(pd2b_ref[...])

    def masked_mean(sa, sb, ca, cb):
        cnt = jnp.sum(ca) + jnp.sum(cb)
        s = jnp.sum(sa) + jnp.sum(sb)
        return jnp.where(cnt > 0, s / jnp.maximum(cnt, 1.0), 0.0)

    csa = csa_ref[...]
    csb = csb_ref[...]
    p_cluster = masked_mean(pda * csa, pdb * csb, csa, csb)
    ma = (dda_ref[...] < 3.0).astype(jnp.float32)
    mb = (ddb_ref[...] < 3.0).astype(jnp.float32)
    p_depth = masked_mean(pda * ma, pdb * mb, ma, mb)

    bits_a = lax.bitcast_convert_type(pda, jnp.int32)
    bits_b = lax.bitcast_convert_type(pdb, jnp.int32)

    def bs_body(_, lohi):
        lo, hi = lohi
        mid = (lo + hi) // 2
        cnt = (jnp.sum((bits_a <= mid).astype(jnp.int32))
               + jnp.sum((bits_b <= mid).astype(jnp.int32)))
        return jnp.where(cnt >= K_RANK, lo, mid + 1), jnp.where(
            cnt >= K_RANK, mid, hi)

    lo, hi = lax.fori_loop(0, 31, bs_body,
                           (jnp.int32(0), jnp.int32(0x7F800000)))
    v1 = lax.bitcast_convert_type(hi, jnp.float32)
    ja = (pda > v1).astype(jnp.float32)
    jb = (pdb > v1).astype(jnp.float32)
    p_jump = masked_mean(pda * ja, pdb * jb, ja, jb)

    e = jnp.exp(raw_ref[...])
    w = e / jnp.sum(e)
    total = w[0, 0] * p_cluster + w[0, 1] * p_depth + w[0, 2] * p_jump
    o_ref[...] = jnp.reshape(total, (1, 1))


def _finalize(pd2a, pd2b, dda, ddb, csa, csb, raw_pad):
    return pl.pallas_call(
        _finalize_body,
        out_shape=jax.ShapeDtypeStruct((1, 1), jnp.float32),
    )(pd2a, pd2b, dda, ddb, csa, csb, raw_pad)


def kernel(logits_src, logits_tgt, edge_index_src, edge_index_tgt,
           cluster_labels_src, cluster_labels_tgt, depth_src, depth_tgt, raw):
    logits = jnp.stack([logits_src, logits_tgt]).astype(jnp.float32)
    probs, norms = _softmax_norms(logits)
    p_pad = jnp.pad(probs, ((0, 0), (0, GW - N), (0, 0)))

    gram_a = _gram(p_pad[0])
    pd2a, dda, csa = _sc_edges(
        gram_a.reshape(N * GWW),
        edge_index_src[0].astype(jnp.int32),
        edge_index_src[1].astype(jnp.int32),
        norms.reshape(2, N)[0], depth_src.astype(jnp.float32),
        cluster_labels_src.astype(jnp.int32))
    gram_b = _gram(p_pad[1])
    pd2b, ddb, csb = _sc_edges(
        gram_b.reshape(N * GWW),
        edge_index_tgt[0].astype(jnp.int32),
        edge_index_tgt[1].astype(jnp.int32),
        norms.reshape(2, N)[1], depth_tgt.astype(jnp.float32),
        cluster_labels_tgt.astype(jnp.int32))

    raw_pad = jnp.full((1, C), -jnp.inf, jnp.float32).at[0, :3].set(
        raw.astype(jnp.float32))
    out = _finalize(pd2a.reshape(2500, 128), pd2b.reshape(2500, 128),
                    dda.reshape(2500, 128), ddb.reshape(2500, 128),
                    csa.reshape(2500, 128), csb.reshape(2500, 128), raw_pad)
    return out[0, 0]
